# baseline (speedup 1.0000x reference)
import jax
import jax.numpy as jnp
from jax.experimental import pallas as pl
from jax.experimental.pallas import tpu as pltpu

_KERNEL_MUL = 2.0
_KERNEL_NUM = 5
_NCHUNK = 4


def _mkmmd_kernel(src_hbm, tgt_hbm, out_ref, buf, sem):
    nb, d = src_hbm.shape
    n = 2 * nb
    ch = nb // _NCHUNK

    def copy(k):
        return (
            pltpu.make_async_copy(src_hbm.at[pl.ds(k * ch, ch)],
                                  buf.at[pl.ds(k * ch, ch)], sem.at[k]),
            pltpu.make_async_copy(tgt_hbm.at[pl.ds(k * ch, ch)],
                                  buf.at[pl.ds(nb + k * ch, ch)],
                                  sem.at[_NCHUNK + k]),
        )

    for k in range(_NCHUNK):
        a, b = copy(k)
        a.start()
        b.start()

    row = jax.lax.broadcasted_iota(jnp.int32, (d, d), 0)
    col = jax.lax.broadcasted_iota(jnp.int32, (d, d), 1)
    tri = jnp.where(row <= col, 1.0, 0.0).astype(jnp.bfloat16)
    w = (d - jax.lax.broadcasted_iota(jnp.int32, (1, d), 1)).astype(jnp.float32)

    for k in range(_NCHUNK):
        a, b = copy(k)
        a.wait()
        b.wait()

    src = buf[:nb]
    tgt = buf[nb:]

    s1 = jnp.sum(src, axis=0, keepdims=True) + jnp.sum(tgt, axis=0, keepdims=True)
    s2 = (jnp.sum(src * src, axis=0, keepdims=True)
          + jnp.sum(tgt * tgt, axis=0, keepdims=True))
    colsum = (2.0 * n) * s2 - 2.0 * s1 * s1
    bw_sum = jnp.sum(w * colsum)
    bw = bw_sum / (n * n - n) / (_KERNEL_MUL ** (_KERNEL_NUM // 2))
    neg_inv = -1.0 / (bw * (_KERNEL_MUL ** (_KERNEL_NUM - 1)))

    rs = jnp.concatenate([src[1:], src[:1]], axis=0)
    rt = jnp.concatenate([tgt[1:], tgt[:1]], axis=0)
    sq = jnp.concatenate(
        [src - rs, tgt - rt, src - rt, rs - tgt], axis=0)
    sq = sq * sq

    hi = sq.astype(jnp.bfloat16)
    lo = (sq - hi.astype(jnp.float32)).astype(jnp.bfloat16)
    c = (jnp.dot(hi, tri, preferred_element_type=jnp.float32)
         + jnp.dot(lo, tri, preferred_element_type=jnp.float32))
    z = jnp.exp(c * neg_inv)
    z2 = z * z
    z4 = z2 * z2
    z8 = z4 * z4
    z16 = z8 * z8
    ksum = z + z2 + z4 + z8 + z16
    comb = ksum[: 2 * nb] - ksum[2 * nb:]

    total = jnp.sum(comb, axis=(0, 1), keepdims=True)
    out_ref[:, :] = total * (1.0 / (nb * d))


@jax.jit
def kernel(source, target):
    b, d = source.shape
    out = pl.pallas_call(
        _mkmmd_kernel,
        out_shape=jax.ShapeDtypeStruct((1, 1), jnp.float32),
        in_specs=[
            pl.BlockSpec(memory_space=pltpu.HBM),
            pl.BlockSpec(memory_space=pltpu.HBM),
        ],
        out_specs=pl.BlockSpec(memory_space=pltpu.VMEM),
        scratch_shapes=[
            pltpu.VMEM((2 * b, d), jnp.float32),
            pltpu.SemaphoreType.DMA((2 * _NCHUNK,)),
        ],
    )(source, target)
    return out[0, 0]


# speedup vs baseline: 1.0114x; 1.0114x over previous
import jax
import jax.numpy as jnp
from jax.experimental import pallas as pl
from jax.experimental.pallas import tpu as pltpu

_KERNEL_MUL = 2.0
_KERNEL_NUM = 5
_NCHUNK = 4


def _mkmmd_kernel(src_hbm, tgt_hbm, out_ref, buf, sem):
    nb, d = src_hbm.shape
    n = 2 * nb
    ch = nb // _NCHUNK

    def copy(k):
        return (
            pltpu.make_async_copy(src_hbm.at[pl.ds(k * ch, ch)],
                                  buf.at[pl.ds(k * ch, ch)], sem),
            pltpu.make_async_copy(tgt_hbm.at[pl.ds(k * ch, ch)],
                                  buf.at[pl.ds(nb + k * ch, ch)],
                                  sem),
        )

    for k in range(_NCHUNK):
        a, b = copy(k)
        a.start()
        b.start()

    row = jax.lax.broadcasted_iota(jnp.int32, (d, d), 0)
    col = jax.lax.broadcasted_iota(jnp.int32, (d, d), 1)
    tri = jnp.where(row <= col, 1.0, 0.0).astype(jnp.bfloat16)
    w = (d - jax.lax.broadcasted_iota(jnp.int32, (1, d), 1)).astype(jnp.float32)

    for k in range(_NCHUNK):
        a, b = copy(k)
        a.wait()
        b.wait()

    src = buf[:nb]
    tgt = buf[nb:]

    s1 = jnp.sum(src, axis=0, keepdims=True) + jnp.sum(tgt, axis=0, keepdims=True)
    s2 = (jnp.sum(src * src, axis=0, keepdims=True)
          + jnp.sum(tgt * tgt, axis=0, keepdims=True))
    colsum = (2.0 * n) * s2 - 2.0 * s1 * s1
    bw_sum = jnp.sum(w * colsum)
    bw = bw_sum / (n * n - n) / (_KERNEL_MUL ** (_KERNEL_NUM // 2))
    neg_inv = -1.0 / (bw * (_KERNEL_MUL ** (_KERNEL_NUM - 1)))

    rs = jnp.concatenate([src[1:], src[:1]], axis=0)
    rt = jnp.concatenate([tgt[1:], tgt[:1]], axis=0)
    sq = jnp.concatenate(
        [src - rs, tgt - rt, src - rt, rs - tgt], axis=0)
    sq = sq * sq

    hi = sq.astype(jnp.bfloat16)
    lo = (sq - hi.astype(jnp.float32)).astype(jnp.bfloat16)
    c = (jnp.dot(hi, tri, preferred_element_type=jnp.float32)
         + jnp.dot(lo, tri, preferred_element_type=jnp.float32))
    z = jnp.exp(c * neg_inv)
    z2 = z * z
    z4 = z2 * z2
    z8 = z4 * z4
    z16 = z8 * z8
    ksum = z + z2 + z4 + z8 + z16
    comb = ksum[: 2 * nb] - ksum[2 * nb:]

    total = jnp.sum(comb, axis=(0, 1), keepdims=True)
    out_ref[:, :] = total * (1.0 / (nb * d))


@jax.jit
def kernel(source, target):
    b, d = source.shape
    out = pl.pallas_call(
        _mkmmd_kernel,
        out_shape=jax.ShapeDtypeStruct((1, 1), jnp.float32),
        in_specs=[
            pl.BlockSpec(memory_space=pltpu.HBM),
            pl.BlockSpec(memory_space=pltpu.HBM),
        ],
        out_specs=pl.BlockSpec(memory_space=pltpu.VMEM),
        scratch_shapes=[
            pltpu.VMEM((2 * b, d), jnp.float32),
            pltpu.SemaphoreType.DMA,
        ],
    )(source, target)
    return out[0, 0]
